# baseline (speedup 1.0000x reference)
import functools

import jax
import jax.numpy as jnp
from jax import lax
from jax.experimental import pallas as pl
from jax.experimental.pallas import tpu as pltpu
from jax.experimental.pallas import tpu_sc as plsc

B = 16384
D = 32
CHUNK = 128

_NC, _NS = 2, 16
_NW = _NC * _NS
_BPW = B // _NW
_NCH = _BPW // CHUNK


def _gather_body(uidx_hbm, midx_hbm, utab_hbm, mtab_hbm, ue_hbm, me_hbm,
                 uidx_v, midx_v, urows_v, mrows_v, sem):
    wid = lax.axis_index("s") * _NC + lax.axis_index("c")
    base = wid * _BPW
    pltpu.sync_copy(uidx_hbm.at[wid], uidx_v)
    pltpu.sync_copy(midx_hbm.at[wid], midx_v)
    copies = []
    for j in range(_NCH):
        copies.append(pltpu.async_copy(
            utab_hbm.at[uidx_v.at[j]], urows_v.at[pl.ds(j * CHUNK, CHUNK)], sem))
        copies.append(pltpu.async_copy(
            mtab_hbm.at[midx_v.at[j]], mrows_v.at[pl.ds(j * CHUNK, CHUNK)], sem))
    for c in copies:
        c.wait()
    pltpu.sync_copy(urows_v, ue_hbm.at[pl.ds(base, _BPW)])
    pltpu.sync_copy(mrows_v, me_hbm.at[pl.ds(base, _BPW)])


@functools.cache
def _gather():
    return pl.kernel(
        _gather_body,
        mesh=plsc.VectorSubcoreMesh(core_axis_name="c", subcore_axis_name="s"),
        out_type=(
            jax.ShapeDtypeStruct((B, D), jnp.float32),
            jax.ShapeDtypeStruct((B, D), jnp.float32),
        ),
        scratch_types=[
            pltpu.VMEM((_NCH, CHUNK), jnp.int32),
            pltpu.VMEM((_NCH, CHUNK), jnp.int32),
            pltpu.VMEM((_BPW, D), jnp.float32),
            pltpu.VMEM((_BPW, D), jnp.float32),
            pltpu.SemaphoreType.DMA,
        ],
        compiler_params=pltpu.CompilerParams(use_tc_tiling_on_sc=False),
    )


def _mlp_body(ue, me, w1u, w1m, b1, w2, b2, w3, b3, out):
    h = jnp.dot(ue[...], w1u[...], preferred_element_type=jnp.float32)
    h = h + jnp.dot(me[...], w1m[...], preferred_element_type=jnp.float32)
    h = jnp.maximum(h + b1[...], 0.0)
    h = jnp.maximum(jnp.dot(h, w2[...], preferred_element_type=jnp.float32) + b2[...], 0.0)
    out[...] = jnp.dot(h, w3[...], preferred_element_type=jnp.float32) + b3[...]


def _mlp(ue, me, w1u, w1m, b1, w2, b2, w3, b3):
    blk = 2048
    rep = lambda i: (0, 0)
    return pl.pallas_call(
        _mlp_body,
        grid=(B // blk,),
        in_specs=[
            pl.BlockSpec((blk, D), lambda i: (i, 0)),
            pl.BlockSpec((blk, D), lambda i: (i, 0)),
            pl.BlockSpec((D, 256), rep),
            pl.BlockSpec((D, 256), rep),
            pl.BlockSpec((1, 256), rep),
            pl.BlockSpec((256, 64), rep),
            pl.BlockSpec((1, 64), rep),
            pl.BlockSpec((64, 1), rep),
            pl.BlockSpec((1, 1), rep),
        ],
        out_specs=pl.BlockSpec((blk, 1), lambda i: (i, 0)),
        out_shape=jax.ShapeDtypeStruct((B, 1), jnp.float32),
    )(ue, me, w1u, w1m, b1, w2, b2, w3, b3)


def kernel(user_id, movie_title, user_table, movie_table, W1, b1, W2, b2, W3, b3):
    uidx = user_id.astype(jnp.int32).reshape(_NW, _NCH, CHUNK)
    midx = movie_title.astype(jnp.int32).reshape(_NW, _NCH, CHUNK)
    ue, me = _gather()(uidx, midx, user_table, movie_table)
    return _mlp(ue, me, W1[:D], W1[D:], b1.reshape(1, -1),
                W2, b2.reshape(1, -1), W3, b3.reshape(1, 1))
